# Initial kernel scaffold; baseline (speedup 1.0000x reference)
import functools

import jax
import jax.numpy as jnp
from jax import lax
from jax.experimental import pallas as pl
from jax.experimental.pallas import tpu as pltpu
from jax.experimental.pallas import tpu_sc as plsc

VOCAB = 1000000
D = 64
B = 16384
CTX = 4
NEG = 5

NC = 2
NS = 16
NW = NC * NS
B_PER_W = B // NW
CHUNK = 128
NCHUNK = B_PER_W // CHUNK
GROUPS = CHUNK // 16


def _body(ctx_idx_hbm, cen_idx_hbm, neg_idx_hbm, ctx_emb_hbm, cen_emb_hbm,
          pos_hbm, neg_hbm,
          idx_ctx, idx_cen, idx_neg, rows_ctx, rows_cen, rows_neg,
          pos_v, neg_v, sem):
  wid = lax.axis_index("s") * NC + lax.axis_index("c")
  base = wid * B_PER_W

  lanes = lax.iota(jnp.int32, 16)

  for c in range(NCHUNK):
    b0 = base + c * CHUNK
    pltpu.sync_copy(ctx_idx_hbm.at[pl.ds(b0 * CTX, CHUNK * CTX)], idx_ctx)
    pltpu.sync_copy(cen_idx_hbm.at[pl.ds(b0, CHUNK)], idx_cen)
    pltpu.sync_copy(neg_idx_hbm.at[pl.ds(b0 * NEG, CHUNK * NEG)], idx_neg)

    cps = []
    for k in range(CTX):
      cps.append(pltpu.make_async_copy(
          ctx_emb_hbm.at[idx_ctx.at[pl.ds(k * 128, 128)]],
          rows_ctx.at[pl.ds(k * 128, 128)], sem))
    cps.append(pltpu.make_async_copy(
        cen_emb_hbm.at[idx_cen], rows_cen, sem))
    for k in range(NEG):
      cps.append(pltpu.make_async_copy(
          cen_emb_hbm.at[idx_neg.at[pl.ds(k * 128, 128)]],
          rows_neg.at[pl.ds(k * 128, 128)], sem))
    for cp in cps:
      cp.start()
    for cp in cps:
      cp.wait()

    def group_body(g, _):
      bl = g * 16 + lanes
      ctx_rows = bl * CTX
      neg_rows = bl * NEG

      def d_body(d, acc):
        pos_a, n0, n1, n2, n3, n4 = acc
        col = jnp.full((16,), d, jnp.int32)
        v = plsc.load_gather(rows_ctx, [ctx_rows, col])
        v = v + plsc.load_gather(rows_ctx, [ctx_rows + 1, col])
        v = v + plsc.load_gather(rows_ctx, [ctx_rows + 2, col])
        v = v + plsc.load_gather(rows_ctx, [ctx_rows + 3, col])
        u = plsc.load_gather(rows_cen, [bl, col])
        pos_a = pos_a + v * u
        n0 = n0 + v * plsc.load_gather(rows_neg, [neg_rows, col])
        n1 = n1 + v * plsc.load_gather(rows_neg, [neg_rows + 1, col])
        n2 = n2 + v * plsc.load_gather(rows_neg, [neg_rows + 2, col])
        n3 = n3 + v * plsc.load_gather(rows_neg, [neg_rows + 3, col])
        n4 = n4 + v * plsc.load_gather(rows_neg, [neg_rows + 4, col])
        return pos_a, n0, n1, n2, n3, n4

      z = jnp.zeros((16,), jnp.float32)
      pos_a, n0, n1, n2, n3, n4 = lax.fori_loop(
          0, D, d_body, (z, z, z, z, z, z))

      quarter = jnp.float32(0.25)
      pos_v[pl.ds(g * 16, 16)] = pos_a * quarter
      plsc.store_scatter(neg_v, [neg_rows], n0 * quarter)
      plsc.store_scatter(neg_v, [neg_rows + 1], n1 * quarter)
      plsc.store_scatter(neg_v, [neg_rows + 2], n2 * quarter)
      plsc.store_scatter(neg_v, [neg_rows + 3], n3 * quarter)
      plsc.store_scatter(neg_v, [neg_rows + 4], n4 * quarter)
      return 0

    lax.fori_loop(0, GROUPS, group_body, 0)

    pltpu.sync_copy(pos_v, pos_hbm.at[pl.ds(b0, CHUNK)])
    pltpu.sync_copy(neg_v, neg_hbm.at[pl.ds(b0 * NEG, CHUNK * NEG)])


@jax.jit
def _cbow_sc(ctx_idx, cen_idx, neg_idx, context_emb, center_emb):
  mesh = plsc.VectorSubcoreMesh(core_axis_name="c", subcore_axis_name="s")
  kfn = pl.kernel(
      _body,
      out_type=(
          jax.ShapeDtypeStruct((B,), jnp.float32),
          jax.ShapeDtypeStruct((B * NEG,), jnp.float32),
      ),
      mesh=mesh,
      scratch_types=[
          pltpu.VMEM((CHUNK * CTX,), jnp.int32),
          pltpu.VMEM((CHUNK,), jnp.int32),
          pltpu.VMEM((CHUNK * NEG,), jnp.int32),
          pltpu.VMEM((CHUNK * CTX, D), jnp.float32),
          pltpu.VMEM((CHUNK, D), jnp.float32),
          pltpu.VMEM((CHUNK * NEG, D), jnp.float32),
          pltpu.VMEM((CHUNK,), jnp.float32),
          pltpu.VMEM((CHUNK * NEG,), jnp.float32),
          pltpu.SemaphoreType.DMA,
      ],
  )
  return kfn(ctx_idx, cen_idx, neg_idx, context_emb, center_emb)


def kernel(context_words, center_words, negative_samples, context_emb,
           center_emb):
  ctx_idx = context_words.reshape(-1).astype(jnp.int32)
  cen_idx = center_words.astype(jnp.int32)
  neg_idx = negative_samples.reshape(-1).astype(jnp.int32)
  pos, neg = _cbow_sc(ctx_idx, cen_idx, neg_idx, context_emb, center_emb)
  return pos, neg.reshape(B, NEG)


# speedup vs baseline: 1.5634x; 1.5634x over previous
import functools

import jax
import jax.numpy as jnp
from jax import lax
from jax.experimental import pallas as pl
from jax.experimental.pallas import tpu as pltpu
from jax.experimental.pallas import tpu_sc as plsc

VOCAB = 1000000
D = 64
B = 16384
CTX = 4
NEG = 5

NC = 2
NS = 16
NW = NC * NS
B_PER_W = B // NW
CHUNK = 128
NCHUNK = B_PER_W // CHUNK
GROUPS = CHUNK // 16


def _body(ctx_idx_hbm, cen_idx_hbm, neg_idx_hbm, ctx_emb_hbm, cen_emb_hbm,
          pos_hbm, neg_hbm,
          idx_ctx, idx_cen, idx_neg, rows_ctx, rows_cen, rows_neg,
          pos_v, neg_v, sem):
  wid = lax.axis_index("s") * NC + lax.axis_index("c")
  base = wid * B_PER_W

  lanes = lax.iota(jnp.int32, 16)

  for c in range(NCHUNK):
    b0 = base + c * CHUNK
    pltpu.sync_copy(ctx_idx_hbm.at[pl.ds(b0 * CTX, CHUNK * CTX)], idx_ctx)
    pltpu.sync_copy(cen_idx_hbm.at[pl.ds(b0, CHUNK)], idx_cen)
    pltpu.sync_copy(neg_idx_hbm.at[pl.ds(b0 * NEG, CHUNK * NEG)], idx_neg)

    cps = []
    for k in range(CTX):
      cps.append(pltpu.make_async_copy(
          ctx_emb_hbm.at[idx_ctx.at[pl.ds(k * 128, 128)]],
          rows_ctx.at[pl.ds(k * 128, 128)], sem))
    cps.append(pltpu.make_async_copy(
        cen_emb_hbm.at[idx_cen], rows_cen, sem))
    for k in range(NEG):
      cps.append(pltpu.make_async_copy(
          cen_emb_hbm.at[idx_neg.at[pl.ds(k * 128, 128)]],
          rows_neg.at[pl.ds(k * 128, 128)], sem))
    for cp in cps:
      cp.start()
    for cp in cps:
      cp.wait()

    def group_body(g, _):
      bl = g * 16 + lanes
      ctx_rows = bl * CTX
      neg_rows = bl * NEG

      def d_body(d, acc):
        pos_a, n0, n1, n2, n3, n4 = acc
        col = jnp.full((16,), d, jnp.int32)
        v = plsc.load_gather(rows_ctx, [ctx_rows, col])
        v = v + plsc.load_gather(rows_ctx, [ctx_rows + 1, col])
        v = v + plsc.load_gather(rows_ctx, [ctx_rows + 2, col])
        v = v + plsc.load_gather(rows_ctx, [ctx_rows + 3, col])
        u = plsc.load_gather(rows_cen, [bl, col])
        pos_a = pos_a + v * u
        n0 = n0 + v * plsc.load_gather(rows_neg, [neg_rows, col])
        n1 = n1 + v * plsc.load_gather(rows_neg, [neg_rows + 1, col])
        n2 = n2 + v * plsc.load_gather(rows_neg, [neg_rows + 2, col])
        n3 = n3 + v * plsc.load_gather(rows_neg, [neg_rows + 3, col])
        n4 = n4 + v * plsc.load_gather(rows_neg, [neg_rows + 4, col])
        return pos_a, n0, n1, n2, n3, n4

      z = jnp.zeros((16,), jnp.float32)
      pos_a, n0, n1, n2, n3, n4 = lax.fori_loop(
          0, D, d_body, (z, z, z, z, z, z))

      quarter = jnp.float32(0.25)
      pos_v[pl.ds(g * 16, 16)] = pos_a * quarter
      plsc.store_scatter(neg_v, [neg_rows], n0 * quarter)
      plsc.store_scatter(neg_v, [neg_rows + 1], n1 * quarter)
      plsc.store_scatter(neg_v, [neg_rows + 2], n2 * quarter)
      plsc.store_scatter(neg_v, [neg_rows + 3], n3 * quarter)
      plsc.store_scatter(neg_v, [neg_rows + 4], n4 * quarter)
      return 0

    lax.fori_loop(0, GROUPS, group_body, 0)

    pltpu.sync_copy(pos_v, pos_hbm.at[pl.ds(b0, CHUNK)])
    pltpu.sync_copy(neg_v, neg_hbm.at[pl.ds(b0 * NEG, CHUNK * NEG)])


@jax.jit
def _cbow_sc(ctx_idx, cen_idx, neg_idx, context_emb, center_emb):
  mesh = plsc.VectorSubcoreMesh(core_axis_name="c", subcore_axis_name="s")
  kfn = pl.kernel(
      _body,
      out_type=(
          jax.ShapeDtypeStruct((B,), jnp.float32),
          jax.ShapeDtypeStruct((B * NEG,), jnp.float32),
      ),
      mesh=mesh,
      compiler_params=pltpu.CompilerParams(needs_layout_passes=False, use_tc_tiling_on_sc=False),
      scratch_types=[
          pltpu.VMEM((CHUNK * CTX,), jnp.int32),
          pltpu.VMEM((CHUNK,), jnp.int32),
          pltpu.VMEM((CHUNK * NEG,), jnp.int32),
          pltpu.VMEM((CHUNK * CTX, D), jnp.float32),
          pltpu.VMEM((CHUNK, D), jnp.float32),
          pltpu.VMEM((CHUNK * NEG, D), jnp.float32),
          pltpu.VMEM((CHUNK,), jnp.float32),
          pltpu.VMEM((CHUNK * NEG,), jnp.float32),
          pltpu.SemaphoreType.DMA,
      ],
  )
  return kfn(ctx_idx, cen_idx, neg_idx, context_emb, center_emb)


def kernel(context_words, center_words, negative_samples, context_emb,
           center_emb):
  ctx_idx = context_words.reshape(-1).astype(jnp.int32)
  cen_idx = center_words.astype(jnp.int32)
  neg_idx = negative_samples.reshape(-1).astype(jnp.int32)
  pos, neg = _cbow_sc(ctx_idx, cen_idx, neg_idx, context_emb, center_emb)
  return pos, neg.reshape(B, NEG)
